# Initial kernel scaffold; baseline (speedup 1.0000x reference)
import jax
import jax.numpy as jnp
from jax.experimental import pallas as pl


def kernel(x, edge_index, W, b, prelu_weight):
    raise NotImplementedError("write your pallas kernel here")


# speedup vs baseline: 26.7575x; 26.7575x over previous
import functools

import jax
import jax.numpy as jnp
from jax import lax
from jax.experimental import pallas as pl
from jax.experimental.pallas import tpu as pltpu
from jax.experimental.pallas import tpu_sc as plsc

N = 10000
E = 320000
D = 128

NC = 2
NS = 16
NW = NC * NS
EPW = E // NW
K = 80
NWIN = EPW // K

ROWS_PER_TILE = N // NS
ZCHUNK = 25

_MESH = plsc.VectorSubcoreMesh(core_axis_name="c", subcore_axis_name="s")


def _fill_i32(ref, n, value):
    v = jnp.full((16,), value, jnp.int32)

    def body(i, _):
        ref[pl.ds(i * 16, 16)] = v
        return 0

    lax.fori_loop(0, n // 16, body, 0)


def _zero_f32_2d(ref, rows, cols):
    z = jnp.zeros((16,), jnp.float32)

    def body(i, _):
        r = i // (cols // 16)
        c = (i % (cols // 16)) * 16
        ref[r, pl.ds(c, 16)] = z
        return 0

    lax.fori_loop(0, rows * (cols // 16), body, 0)


def _sc_degree_body(dst_hbm, out_hbm, idx_v, ones_v, zeros_v, hist_sh):
    c = lax.axis_index("c")
    s = lax.axis_index("s")
    wid = c * NS + s

    @pl.when(s == 0)
    def _():
        _fill_i32(zeros_v, N, 0)
        pltpu.sync_copy(zeros_v, hist_sh)

    _fill_i32(ones_v, K, 1)
    pltpu.sync_copy(dst_hbm.at[wid], idx_v)
    plsc.subcore_barrier()

    def win(j, _):
        pltpu.sync_copy(ones_v, hist_sh.at[idx_v.at[j]], add=True)
        return 0

    lax.fori_loop(0, NWIN, win, 0)
    plsc.subcore_barrier()

    @pl.when(s == 0)
    def _():
        pltpu.sync_copy(hist_sh, out_hbm.at[c])


_sc_degree = pl.kernel(
    _sc_degree_body,
    out_type=jax.ShapeDtypeStruct((NC, N), jnp.int32),
    mesh=_MESH,
    scratch_types=[
        pltpu.VMEM((NWIN, K), jnp.int32),
        pltpu.VMEM((K,), jnp.int32),
        pltpu.VMEM((N,), jnp.int32),
        pltpu.VMEM_SHARED((N,), jnp.int32),
    ],
)


_BN = 1000


def _tc_h2_body(hist_ref, x_ref, w_ref, h2_ref):
    deg = (hist_ref[:, 0] + hist_ref[:, 1] + 1).astype(jnp.float32)
    dis = lax.rsqrt(deg)
    h = jnp.dot(x_ref[...], w_ref[...], preferred_element_type=jnp.float32)
    h2_ref[...] = h * dis[:, None]


def _tc_h2(histT, x, W):
    return pl.pallas_call(
        _tc_h2_body,
        grid=(N // _BN,),
        in_specs=[
            pl.BlockSpec((_BN, NC), lambda i: (i, 0)),
            pl.BlockSpec((_BN, D), lambda i: (i, 0)),
            pl.BlockSpec((D, D), lambda i: (0, 0)),
        ],
        out_specs=pl.BlockSpec((_BN, D), lambda i: (i, 0)),
        out_shape=jax.ShapeDtypeStruct((N, D), jnp.float32),
    )(histT, x, W)


def _sc_scatter_body(h2_hbm, src_hbm, dst_hbm, out_hbm,
                     sidx_v, didx_v, rows_v, a_sh, sem):
    c = lax.axis_index("c")
    s = lax.axis_index("s")
    wid = c * NS + s

    _zero_f32_2d(rows_v, ZCHUNK, D)

    def zloop(k, _):
        pltpu.sync_copy(rows_v.at[pl.ds(0, ZCHUNK)],
                        a_sh.at[pl.ds(s * ROWS_PER_TILE + k * ZCHUNK, ZCHUNK)])
        return 0

    lax.fori_loop(0, ROWS_PER_TILE // ZCHUNK, zloop, 0)

    pltpu.sync_copy(src_hbm.at[wid], sidx_v)
    pltpu.sync_copy(dst_hbm.at[wid], didx_v)
    plsc.subcore_barrier()

    def win(j, _):
        pltpu.async_copy(h2_hbm.at[sidx_v.at[j]], rows_v, sem).wait()
        pltpu.sync_copy(rows_v, a_sh.at[didx_v.at[j]], add=True)
        return 0

    lax.fori_loop(0, NWIN, win, 0)
    plsc.subcore_barrier()

    pltpu.sync_copy(a_sh.at[pl.ds(s * ROWS_PER_TILE, ROWS_PER_TILE)],
                    out_hbm.at[c, s])


_sc_scatter = pl.kernel(
    _sc_scatter_body,
    out_type=jax.ShapeDtypeStruct((NC, NS, ROWS_PER_TILE, D), jnp.float32),
    mesh=_MESH,
    scratch_types=[
        pltpu.VMEM((NWIN, K), jnp.int32),
        pltpu.VMEM((NWIN, K), jnp.int32),
        pltpu.VMEM((K, D), jnp.float32),
        pltpu.VMEM_SHARED((N, D), jnp.float32),
        pltpu.SemaphoreType.DMA,
    ],
)


def _tc_out_body(hist_ref, a_ref, h2_ref, b_ref, pw_ref, o_ref):
    deg = (hist_ref[:, 0] + hist_ref[:, 1] + 1).astype(jnp.float32)
    dis = lax.rsqrt(deg)
    z = (a_ref[0] + a_ref[1] + h2_ref[...]) * dis[:, None] + b_ref[...]
    o_ref[...] = jnp.where(z >= 0, z, z * pw_ref[...])


def _tc_out(histT, a, h2, b2, pw2):
    return pl.pallas_call(
        _tc_out_body,
        grid=(N // _BN,),
        in_specs=[
            pl.BlockSpec((_BN, NC), lambda i: (i, 0)),
            pl.BlockSpec((NC, _BN, D), lambda i: (0, i, 0)),
            pl.BlockSpec((_BN, D), lambda i: (i, 0)),
            pl.BlockSpec((1, D), lambda i: (0, 0)),
            pl.BlockSpec((1, D), lambda i: (0, 0)),
        ],
        out_specs=pl.BlockSpec((_BN, D), lambda i: (i, 0)),
        out_shape=jax.ShapeDtypeStruct((N, D), jnp.float32),
    )(histT, a, h2, b2, pw2)


def kernel(x, edge_index, W, b, prelu_weight):
    ei = edge_index.astype(jnp.int32)
    src3 = ei[0].reshape(NW, NWIN, K)
    dst3 = ei[1].reshape(NW, NWIN, K)
    hist = _sc_degree(dst3)
    histT = hist.T
    h2 = _tc_h2(histT, x, W)
    a = _sc_scatter(h2, src3, dst3).reshape(NC, N, D)
    return _tc_out(histT, a, h2, b.reshape(1, D), prelu_weight.reshape(1, D))
